# baseline (speedup 1.0000x reference)
import jax
import jax.numpy as jnp
from jax import lax
from jax.experimental import pallas as pl
from jax.experimental.pallas import tpu as pltpu
from jax.experimental.pallas import tpu_sc as plsc

_NW = 16
_NJ = 3
_CHUNK = 38
_GS = (64, 32, 16)
_C = 85


def _sc_body(p0, p1, p2, tg, an0, an1, an2, out,
             tgt_v, anch_v, idx_v, dst0_v, dst1_v, dst2_v, stage_v,
             stg_sem, sem0, sem1, sem2):
    preds = (p0, p1, p2)
    dsts = (dst0_v, dst1_v, dst2_v)
    sems = (sem0, sem1, sem2)

    wid = lax.axis_index("s")

    stg = [pltpu.async_copy(tg, tgt_v, stg_sem),
           pltpu.async_copy(an0, anch_v.at[pl.ds(0, 6)], stg_sem),
           pltpu.async_copy(an1, anch_v.at[pl.ds(8, 6)], stg_sem),
           pltpu.async_copy(an2, anch_v.at[pl.ds(16, 6)], stg_sem)]
    for cp in stg:
        cp.wait()

    base = wid * _CHUNK
    lane = lax.iota(jnp.int32, 16)

    av, validv = [], []
    bfv, xv, yv, wv, hv = [], [], [], [], []
    for j in range(_NJ):
        p = base + j * 16 + lane
        valid = jnp.logical_and(j * 16 + lane < _CHUNK, p < 600)
        a = lax.div(p, jnp.int32(200))
        n = lax.rem(p, jnp.int32(200))
        av.append(a); validv.append(valid)
        n6 = n * 6
        col = lambda c: plsc.load_gather(tgt_v, [n6 + c])
        bfv.append(col(0))
        xv.append(col(2)); yv.append(col(3))
        wv.append(col(4)); hv.append(col(5))

    for l in range(3):
        G = _GS[l]
        for j in range(_NJ):
            gi = jnp.clip((xv[j] * float(G)).astype(jnp.int32), 0, G - 1)
            gj = jnp.clip((yv[j] * float(G)).astype(jnp.int32), 0, G - 1)
            b = bfv[j].astype(jnp.int32)
            row = ((b * 3 + av[j]) * G + gj) * G + gi
            idx_v[pl.ds(l * 48 + j * 16, 16)] = jnp.where(validv[j], row, 0)
    for l in range(3):
        def enqueue(k, carry, l=l):
            r = plsc.load_gather(
                idx_v, [jnp.full((16,), l * 48, jnp.int32) + k])[0]
            pltpu.async_copy(
                preds[l].at[pl.ds(r, 1), :],
                dsts[l].at[pl.ds(k, 1), :],
                sems[l])
            return carry
        lax.fori_loop(0, _CHUNK, enqueue, jnp.int32(0))

    for l in range(3):
        G = _GS[l]
        gf = float(G)
        pltpu.make_async_copy(
            preds[l].at[pl.ds(0, 32), :],
            dsts[l].at[pl.ds(0, 32), :], sems[l]).wait()
        for k in range(32, _CHUNK):
            pltpu.make_async_copy(
                preds[l].at[pl.ds(0, 1), :],
                dsts[l].at[pl.ds(k, 1), :], sems[l]).wait()
        acc_s = jnp.zeros((16,), jnp.float32)
        acc_c = jnp.zeros((16,), jnp.float32)
        for j in range(_NJ):
            gx = xv[j] * gf
            gy = yv[j] * gf
            gw = wv[j] * gf
            gh = hv[j] * gf
            tbx = gx - gx.astype(jnp.int32).astype(jnp.float32)
            tby = gy - gy.astype(jnp.int32).astype(jnp.float32)
            amin = jnp.minimum(av[j], 2)
            abase = l * 8 + amin * 2
            aw = plsc.load_gather(anch_v, [abase])
            ah = plsc.load_gather(anch_v, [abase + 1])
            rw = gw / aw
            rh = gh / ah
            rmax = jnp.maximum(jnp.maximum(rw, 1.0 / rw),
                               jnp.maximum(rh, 1.0 / rh))
            jf = jnp.where(jnp.logical_and(validv[j], rmax < 4.0), 1.0, 0.0)
            kidx = jnp.minimum(j * 16 + lane, _CHUNK - 1)
            psc = lambda c: plsc.load_gather(
                dsts[l], [kidx, jnp.full((16,), c, jnp.int32)])
            ps0 = psc(0)
            ps1 = psc(1)
            ps2 = psc(2)
            ps3 = psc(3)
            px = 1.0 / (1.0 + jnp.exp(-ps0))
            py = 1.0 / (1.0 + jnp.exp(-ps1))
            pw = jnp.exp(ps2) * aw
            ph = jnp.exp(ps3) * ah
            iw = jnp.maximum(
                jnp.minimum(px + pw * 0.5, tbx + gw * 0.5)
                - jnp.maximum(px - pw * 0.5, tbx - gw * 0.5), 0.0)
            ih = jnp.maximum(
                jnp.minimum(py + ph * 0.5, tby + gh * 0.5)
                - jnp.maximum(py - ph * 0.5, tby - gh * 0.5), 0.0)
            inter = iw * ih
            union = pw * ph + gw * gh - inter + 1e-7
            iou = inter / union
            acc_s = acc_s + (1.0 - iou) * jf
            acc_c = acc_c + jf
        stage_v[pl.ds(l * 16, 16)] = acc_s
        stage_v[pl.ds((3 + l) * 16, 16)] = acc_c

    pltpu.sync_copy(stage_v, out.at[wid])


@jax.jit
def _sc_partials(p0, p1, p2, t1d, a0, a1, a2):
    mesh = plsc.VectorSubcoreMesh(core_axis_name="c", subcore_axis_name="s", num_cores=1)
    fn = pl.kernel(
        _sc_body,
        mesh=mesh,
        compiler_params=pltpu.CompilerParams(needs_layout_passes=False),
        out_type=jax.ShapeDtypeStruct((_NW, 96), jnp.float32),
        scratch_types=[
            pltpu.VMEM((1200,), jnp.float32),
            pltpu.VMEM((24,), jnp.float32),
            pltpu.VMEM((144,), jnp.int32),
            pltpu.VMEM((_CHUNK, _C), jnp.float32),
            pltpu.VMEM((_CHUNK, _C), jnp.float32),
            pltpu.VMEM((_CHUNK, _C), jnp.float32),
            pltpu.VMEM((96,), jnp.float32),
            pltpu.SemaphoreType.DMA,
            pltpu.SemaphoreType.DMA,
            pltpu.SemaphoreType.DMA,
            pltpu.SemaphoreType.DMA,
        ],
    )
    return fn(p0, p1, p2, t1d, a0, a1, a2)


def kernel(pred0, pred1, pred2, targets, anchors0, anchors1, anchors2):
    p0 = pred0.reshape(-1, _C)
    p1 = pred1.reshape(-1, _C)
    p2 = pred2.reshape(-1, _C)
    parts = _sc_partials(
        p0, p1, p2, targets.reshape(1200), anchors0.reshape(6),
        anchors1.reshape(6), anchors2.reshape(6))
    t = parts.reshape(_NW, 6, 16).sum(axis=(0, 2))
    lbox = (t[0] / jnp.maximum(t[3], 1.0)
            + t[1] / jnp.maximum(t[4], 1.0)
            + t[2] / jnp.maximum(t[5], 1.0))
    return lbox.astype(jnp.float32)


# speedup vs baseline: 1.0513x; 1.0513x over previous
import jax
import jax.numpy as jnp
from jax import lax
from jax.experimental import pallas as pl
from jax.experimental.pallas import tpu as pltpu
from jax.experimental.pallas import tpu_sc as plsc

_NW = 16
_NJ = 3
_CHUNK = 38
_GS = (64, 32, 16)
_C = 85


def _sc_body(p0, p1, p2, tg, out,
             tgt_v, idx_v, dst0_v, dst1_v, dst2_v, stage_v,
             stg_sem, sem0, sem1, sem2):
    preds = (p0, p1, p2)
    dsts = (dst0_v, dst1_v, dst2_v)
    sems = (sem0, sem1, sem2)

    wid = lax.axis_index("s")

    pltpu.async_copy(tg, tgt_v, stg_sem).wait()

    base = wid * _CHUNK
    lane = lax.iota(jnp.int32, 16)

    av, validv = [], []
    bfv, xv, yv, wv, hv = [], [], [], [], []
    for j in range(_NJ):
        p = base + j * 16 + lane
        valid = jnp.logical_and(j * 16 + lane < _CHUNK, p < 600)
        a = lax.div(p, jnp.int32(200))
        n = lax.rem(p, jnp.int32(200))
        av.append(a); validv.append(valid)
        n6 = n * 6
        col = lambda c: plsc.load_gather(tgt_v, [n6 + c])
        bfv.append(col(0))
        xv.append(col(2)); yv.append(col(3))
        wv.append(col(4)); hv.append(col(5))

    for l in range(3):
        G = _GS[l]
        for j in range(_NJ):
            gi = jnp.clip((xv[j] * float(G)).astype(jnp.int32), 0, G - 1)
            gj = jnp.clip((yv[j] * float(G)).astype(jnp.int32), 0, G - 1)
            b = bfv[j].astype(jnp.int32)
            row = ((b * 3 + av[j]) * G + gj) * G + gi
            idx_v[pl.ds(l * 48 + j * 16, 16)] = jnp.where(validv[j], row, 0)
    for l in range(3):
        def enqueue(k, carry, l=l):
            r = plsc.load_gather(
                idx_v, [jnp.full((16,), l * 48, jnp.int32) + k])[0]
            pltpu.async_copy(
                preds[l].at[pl.ds(r, 1), :],
                dsts[l].at[pl.ds(k, 1), :],
                sems[l])
            return carry
        lax.fori_loop(0, _CHUNK, enqueue, jnp.int32(0))

    for l in range(3):
        G = _GS[l]
        gf = float(G)
        pltpu.make_async_copy(
            preds[l].at[pl.ds(0, 32), :],
            dsts[l].at[pl.ds(0, 32), :], sems[l]).wait()
        for k in range(32, _CHUNK):
            pltpu.make_async_copy(
                preds[l].at[pl.ds(0, 1), :],
                dsts[l].at[pl.ds(k, 1), :], sems[l]).wait()
        acc_s = jnp.zeros((16,), jnp.float32)
        acc_c = jnp.zeros((16,), jnp.float32)
        for j in range(_NJ):
            gx = xv[j] * gf
            gy = yv[j] * gf
            gw = wv[j] * gf
            gh = hv[j] * gf
            tbx = gx - gx.astype(jnp.int32).astype(jnp.float32)
            tby = gy - gy.astype(jnp.int32).astype(jnp.float32)
            amin = jnp.minimum(av[j], 2)
            abase = 1200 + l * 8 + amin * 2
            aw = plsc.load_gather(tgt_v, [abase])
            ah = plsc.load_gather(tgt_v, [abase + 1])
            rw = gw / aw
            rh = gh / ah
            rmax = jnp.maximum(jnp.maximum(rw, 1.0 / rw),
                               jnp.maximum(rh, 1.0 / rh))
            jf = jnp.where(jnp.logical_and(validv[j], rmax < 4.0), 1.0, 0.0)
            kidx = jnp.minimum(j * 16 + lane, _CHUNK - 1)
            psc = lambda c: plsc.load_gather(
                dsts[l], [kidx, jnp.full((16,), c, jnp.int32)])
            ps0 = psc(0)
            ps1 = psc(1)
            ps2 = psc(2)
            ps3 = psc(3)
            px = 1.0 / (1.0 + jnp.exp(-ps0))
            py = 1.0 / (1.0 + jnp.exp(-ps1))
            pw = jnp.exp(ps2) * aw
            ph = jnp.exp(ps3) * ah
            iw = jnp.maximum(
                jnp.minimum(px + pw * 0.5, tbx + gw * 0.5)
                - jnp.maximum(px - pw * 0.5, tbx - gw * 0.5), 0.0)
            ih = jnp.maximum(
                jnp.minimum(py + ph * 0.5, tby + gh * 0.5)
                - jnp.maximum(py - ph * 0.5, tby - gh * 0.5), 0.0)
            inter = iw * ih
            union = pw * ph + gw * gh - inter + 1e-7
            iou = inter / union
            acc_s = acc_s + (1.0 - iou) * jf
            acc_c = acc_c + jf
        stage_v[pl.ds(l * 16, 16)] = acc_s
        stage_v[pl.ds((3 + l) * 16, 16)] = acc_c

    pltpu.sync_copy(stage_v, out.at[wid])


@jax.jit
def _sc_partials(p0, p1, p2, table):
    mesh = plsc.VectorSubcoreMesh(core_axis_name="c", subcore_axis_name="s", num_cores=1)
    fn = pl.kernel(
        _sc_body,
        mesh=mesh,
        compiler_params=pltpu.CompilerParams(needs_layout_passes=False),
        out_type=jax.ShapeDtypeStruct((_NW, 96), jnp.float32),
        scratch_types=[
            pltpu.VMEM((1224,), jnp.float32),
            pltpu.VMEM((144,), jnp.int32),
            pltpu.VMEM((_CHUNK, _C), jnp.float32),
            pltpu.VMEM((_CHUNK, _C), jnp.float32),
            pltpu.VMEM((_CHUNK, _C), jnp.float32),
            pltpu.VMEM((96,), jnp.float32),
            pltpu.SemaphoreType.DMA,
            pltpu.SemaphoreType.DMA,
            pltpu.SemaphoreType.DMA,
            pltpu.SemaphoreType.DMA,
        ],
    )
    return fn(p0, p1, p2, table)


def kernel(pred0, pred1, pred2, targets, anchors0, anchors1, anchors2):
    p0 = pred0.reshape(-1, _C)
    p1 = pred1.reshape(-1, _C)
    p2 = pred2.reshape(-1, _C)
    pad2 = jnp.ones((2,), jnp.float32)
    table = jnp.concatenate(
        [targets.reshape(1200), anchors0.reshape(6), pad2,
         anchors1.reshape(6), pad2, anchors2.reshape(6), pad2])
    parts = _sc_partials(p0, p1, p2, table)
    t = parts.reshape(_NW, 6, 16).sum(axis=(0, 2))
    lbox = (t[0] / jnp.maximum(t[3], 1.0)
            + t[1] / jnp.maximum(t[4], 1.0)
            + t[2] / jnp.maximum(t[5], 1.0))
    return lbox.astype(jnp.float32)
